# baseline (speedup 1.0000x reference)
import jax
import jax.numpy as jnp
from jax.experimental import pallas as pl

_ROWS = 512


def _onehot_argmax_block(x_ref, o_ref):
    x = x_ref[...]
    n = x.shape[1]
    m = jnp.max(x, axis=1, keepdims=True)
    iota = jax.lax.broadcasted_iota(jnp.int32, x.shape, 1)
    idx = jnp.min(jnp.where(x == m, iota, n), axis=1, keepdims=True)
    o_ref[...] = (iota == idx).astype(o_ref.dtype)


def kernel(x):
    b, s, n = x.shape
    rows = b * s
    x2 = x.reshape(rows, n)
    out = pl.pallas_call(
        _onehot_argmax_block,
        grid=(rows // _ROWS,),
        in_specs=[pl.BlockSpec((_ROWS, n), lambda i: (i, 0))],
        out_specs=pl.BlockSpec((_ROWS, n), lambda i: (i, 0)),
        out_shape=jax.ShapeDtypeStruct((rows, n), x.dtype),
    )(x2)
    return out.reshape(b, s, n)


# speedup vs baseline: 1.1833x; 1.1833x over previous
import jax
import jax.numpy as jnp
from jax.experimental import pallas as pl

_ROWS = 2048


def _onehot_argmax_block(x_ref, o_ref):
    x = x_ref[...]
    n = x.shape[1]
    m = jnp.max(x, axis=1, keepdims=True)
    iota = jax.lax.broadcasted_iota(jnp.int32, x.shape, 1)
    idx = jnp.min(jnp.where(x == m, iota, n), axis=1, keepdims=True)
    o_ref[...] = (iota == idx).astype(o_ref.dtype)


def kernel(x):
    b, s, n = x.shape
    rows = b * s
    x2 = x.reshape(rows, n)
    out = pl.pallas_call(
        _onehot_argmax_block,
        grid=(rows // _ROWS,),
        in_specs=[pl.BlockSpec((_ROWS, n), lambda i: (i, 0))],
        out_specs=pl.BlockSpec((_ROWS, n), lambda i: (i, 0)),
        out_shape=jax.ShapeDtypeStruct((rows, n), x.dtype),
    )(x2)
    return out.reshape(b, s, n)


# speedup vs baseline: 1.1974x; 1.0119x over previous
import jax
import jax.numpy as jnp
from jax.experimental import pallas as pl

_ROWS = 3072


def _onehot_argmax_block(x_ref, o_ref):
    x = x_ref[...]
    n = x.shape[1]
    m = jnp.max(x, axis=1, keepdims=True)
    iota = jax.lax.broadcasted_iota(jnp.int32, x.shape, 1)
    idx = jnp.min(jnp.where(x == m, iota, n), axis=1, keepdims=True)
    o_ref[...] = (iota == idx).astype(o_ref.dtype)


def kernel(x):
    b, s, n = x.shape
    rows = b * s
    x2 = x.reshape(rows, n)
    out = pl.pallas_call(
        _onehot_argmax_block,
        grid=(rows // _ROWS,),
        in_specs=[pl.BlockSpec((_ROWS, n), lambda i: (i, 0))],
        out_specs=pl.BlockSpec((_ROWS, n), lambda i: (i, 0)),
        out_shape=jax.ShapeDtypeStruct((rows, n), x.dtype),
    )(x2)
    return out.reshape(b, s, n)
